# baseline (speedup 1.0000x reference)
import functools

import jax
import jax.numpy as jnp
from jax import lax
from jax.experimental import pallas as pl
from jax.experimental.pallas import tpu as pltpu
from jax.experimental.pallas import tpu_sc as plsc

NC = 2
NS = 16
NW = NC * NS
CH = 128
TRASH = 64
LANES = 16
ACT_DIM = 8


def _leaky_relu(x):
    return jnp.where(x > 0, x, 0.01 * x)


@functools.lru_cache(maxsize=None)
def _make_sc_deg(n, nch):
    mesh = plsc.VectorSubcoreMesh(core_axis_name="c", subcore_axis_name="s",
                                  num_cores=NC, num_subcores=NS)
    zch = n // 5

    @functools.partial(
        pl.kernel,
        out_type=jax.ShapeDtypeStruct((NC * n,), jnp.float32),
        mesh=mesh,
        compiler_params=pltpu.CompilerParams(use_tc_tiling_on_sc=False),
        scratch_types=[
            pltpu.VMEM((nch, CH), jnp.int32),
            pltpu.VMEM((CH,), jnp.float32),
            pltpu.VMEM((zch,), jnp.float32),
            pltpu.VMEM_SHARED((n + TRASH,), jnp.float32),
        ],
    )
    def deg_kernel(dst_hbm, out_hbm, dst_v, ones_v, zb_v, deg_sh):
        cid = lax.axis_index("c")
        sid = lax.axis_index("s")
        pltpu.sync_copy(dst_hbm.at[cid, sid], dst_v)

        one16 = jnp.full((LANES,), 1.0, jnp.float32)
        zero16 = jnp.zeros((LANES,), jnp.float32)

        def fill_ones(i, c):
            ones_v[pl.ds(pl.multiple_of(i * LANES, LANES), LANES)] = one16
            return c
        lax.fori_loop(0, CH // LANES, fill_ones, 0)

        def fill_zeros(i, c):
            zb_v[pl.ds(pl.multiple_of(i * LANES, LANES), LANES)] = zero16
            return c
        lax.fori_loop(0, zch // LANES, fill_zeros, 0)

        @pl.when(sid < 5)
        def _():
            pltpu.sync_copy(zb_v, deg_sh.at[pl.ds(sid * zch, zch)])
        plsc.subcore_barrier()

        def body(j, c):
            pltpu.sync_copy(ones_v, deg_sh.at[dst_v.at[j]], add=True)
            return c
        lax.fori_loop(0, nch, body, 0)
        plsc.subcore_barrier()

        @pl.when(sid < 5)
        def _():
            pltpu.sync_copy(deg_sh.at[pl.ds(sid * zch, zch)], zb_v)
            pltpu.sync_copy(zb_v, out_hbm.at[pl.ds(cid * n + sid * zch, zch)])

    return deg_kernel


@functools.lru_cache(maxsize=None)
def _make_sc_msg(n, d, nch):
    hd = d // 2
    mesh = plsc.VectorSubcoreMesh(core_axis_name="c", subcore_axis_name="s",
                                  num_cores=NC, num_subcores=NS)
    io_rows = n // 10
    zrows = io_rows // 5
    NB = 4
    assert nch >= NB

    @functools.partial(
        pl.kernel,
        out_type=jax.ShapeDtypeStruct((NC, n, hd), jnp.float32),
        mesh=mesh,
        compiler_params=pltpu.CompilerParams(use_tc_tiling_on_sc=False),
        scratch_types=[
            pltpu.VMEM((nch, CH), jnp.int32),
            pltpu.VMEM((nch, CH), jnp.int32),
            pltpu.VMEM((4, CH, hd), jnp.float32),
            pltpu.VMEM((zrows, hd), jnp.float32),
            pltpu.VMEM_SHARED((n + TRASH, hd), jnp.float32),
            pltpu.SemaphoreType.DMA,
            pltpu.SemaphoreType.DMA,
            pltpu.SemaphoreType.DMA,
            pltpu.SemaphoreType.DMA,
            pltpu.SemaphoreType.DMA,
            pltpu.SemaphoreType.DMA,
            pltpu.SemaphoreType.DMA,
            pltpu.SemaphoreType.DMA,
        ],
    )
    def msg_kernel(y2_hbm, src_hbm, dst_hbm, out_hbm,
                   src_v, dst_v, rows_v, zb_v, acc_sh,
                   sg0, sg1, sg2, sg3, ss0, ss1, ss2, ss3):
        cid = lax.axis_index("c")
        sid = lax.axis_index("s")
        pltpu.sync_copy(src_hbm.at[sid], src_v)
        pltpu.sync_copy(dst_hbm.at[sid], dst_v)

        zero16 = jnp.zeros((LANES,), jnp.float32)
        dl = hd // LANES

        def zrow(i, c):
            r = i // dl
            col = i % dl
            zb_v[r, pl.ds(pl.multiple_of(col * LANES, LANES), LANES)] = zero16
            return c
        lax.fori_loop(0, zrows * dl, zrow, 0)

        @pl.when(sid < 10)
        def _():
            def zcopy(k, c):
                pltpu.sync_copy(zb_v,
                                acc_sh.at[pl.ds(sid * io_rows + k * zrows,
                                                zrows)])
                return c
            lax.fori_loop(0, io_rows // zrows, zcopy, 0)
        plsc.subcore_barrier()

        sems_g = (sg0, sg1, sg2, sg3)
        sems_s = (ss0, ss1, ss2, ss3)

        def g_start(j, b):
            pltpu.async_copy(y2_hbm.at[cid].at[src_v.at[j]], rows_v.at[b],
                             sems_g[b])

        def g_wait(j, b):
            pltpu.make_async_copy(y2_hbm.at[cid].at[src_v.at[j]],
                                  rows_v.at[b], sems_g[b]).wait()

        def s_start(j, b):
            pltpu.async_copy(rows_v.at[b], acc_sh.at[dst_v.at[j]], sems_s[b],
                             add=True)

        def s_wait(j, b):
            pltpu.make_async_copy(rows_v.at[b], acc_sh.at[dst_v.at[j]],
                                  sems_s[b]).wait()

        def visit(j, b):
            g_wait(j, b)
            s_start(j, b)
            jm = j - 2

            @pl.when(jm >= 0)
            def _():
                s_wait(jm, (b + 2) % NB)
            nxt = j + 2

            @pl.when(nxt < nch)
            def _():
                g_start(nxt, (b + 2) % NB)

        g_start(0, 0)
        g_start(1, 1)
        full = nch // NB

        def body(i, c):
            for b in range(NB):
                visit(i * NB + b, b)
            return c
        lax.fori_loop(0, full, body, 0)
        for t in range(full * NB, nch):
            visit(t, t % NB)
        s_wait(nch - 2, (nch - 2) % NB)
        s_wait(nch - 1, (nch - 1) % NB)
        plsc.subcore_barrier()

        @pl.when(sid < 10)
        def _():
            def wcopy(k, c):
                off = sid * io_rows + k * zrows
                pltpu.sync_copy(acc_sh.at[pl.ds(off, zrows)], zb_v)
                pltpu.sync_copy(zb_v, out_hbm.at[cid, pl.ds(off, zrows)])
                return c
            lax.fori_loop(0, io_rows // zrows, wcopy, 0)

    return msg_kernel


def _tc_mm_call(state, w_conv):
    n, d = state.shape
    hd = d // 2

    def mm(x_ref, w_ref, xw2_ref):
        xw = jnp.dot(x_ref[...], w_ref[...], preferred_element_type=jnp.float32)
        xw2_ref[0] = xw[:, :hd]
        xw2_ref[1] = xw[:, hd:]

    return pl.pallas_call(
        mm,
        out_shape=jax.ShapeDtypeStruct((NC, n, hd), jnp.float32),
    )(state, w_conv)


def _tc_scale_call(cnt, xw2):
    _, n, hd = xw2.shape

    def sc(cnt_ref, xw2_ref, y2_ref, dinv_ref):
        deg = cnt_ref[0] + cnt_ref[1] + 1.0
        dinv = lax.rsqrt(deg)
        y2_ref[0] = xw2_ref[0] * dinv[:, None]
        y2_ref[1] = xw2_ref[1] * dinv[:, None]
        dinv_ref[...] = dinv

    return pl.pallas_call(
        sc,
        out_shape=[jax.ShapeDtypeStruct((NC, n, hd), jnp.float32),
                   jax.ShapeDtypeStruct((n,), jnp.float32)],
    )(cnt, xw2)


def _tc_final_call(partials, dinv, state, b_conv, w1, b1, w2, b2, w3r, b3):
    n, d = state.shape

    def fin(p_ref, dinv_ref, x_ref, bc_ref, w1_ref, b1_ref, w2_ref,
            b2_ref, w3_ref, b3_ref, out_ref):
        acc = jnp.concatenate([p_ref[0], p_ref[1]], axis=-1)
        conv = acc * dinv_ref[...][:, None] + bc_ref[...][None, :]
        x = jnp.maximum(conv, 0.0) + x_ref[...]
        h1 = _leaky_relu(jnp.dot(x, w1_ref[...],
                                 preferred_element_type=jnp.float32)
                         + b1_ref[...][None, :])
        h2 = _leaky_relu(jnp.dot(h1, w2_ref[...],
                                 preferred_element_type=jnp.float32)
                         + b2_ref[...][None, :])
        logit = jnp.sum(h2 * w3_ref[...], axis=1) + b3_ref[0]
        conc = jnp.log(1.0 + jnp.exp(-jnp.abs(logit))) + jnp.maximum(logit, 0.0)
        out_ref[...] = conc / (jnp.sum(conc) + 1e-20)

    return pl.pallas_call(
        fin,
        out_shape=jax.ShapeDtypeStruct((n,), jnp.float32),
    )(partials, dinv, state, b_conv, w1, b1, w2, b2, w3r, b3)


def kernel(state, edge_index, W_conv, b_conv, W1, b1, W2, b2, W3, b3,
           deterministic):
    n, d = state.shape
    e = edge_index.shape[1]

    def pad_edges(arr, parts):
        srcp, dstp = arr
        per = srcp.shape[0] // parts
        nchunks = -(-per // CH)
        pad = nchunks * CH - per
        srcp = srcp.reshape(parts, per)
        dstp = dstp.reshape(parts, per)
        if pad:
            k = jnp.arange(pad, dtype=jnp.int32)
            ps = jnp.broadcast_to((k * 131) % n, (parts, pad))
            pd = jnp.broadcast_to(n + (k % TRASH), (parts, pad))
            srcp = jnp.concatenate([srcp, ps], axis=1)
            dstp = jnp.concatenate([dstp, pd], axis=1)
        return (srcp.reshape(parts, nchunks, CH),
                dstp.reshape(parts, nchunks, CH), nchunks)

    _, dstd, nch = pad_edges((edge_index[0], edge_index[1]), NW)
    loop = jnp.arange(n, dtype=edge_index.dtype)
    srcl = jnp.concatenate([edge_index[0], loop])
    dstl = jnp.concatenate([edge_index[1], loop])
    src2, dst2, nch2 = pad_edges((srcl, dstl), NS)
    dstd = dstd.reshape(NC, NS, nch, CH)

    xw2 = _tc_mm_call(state, W_conv)
    cnt = _make_sc_deg(n, nch)(dstd).reshape(NC, n)
    y2, dinv = _tc_scale_call(cnt, xw2)
    partials = _make_sc_msg(n, d, nch2)(y2, src2, dst2)
    act = _tc_final_call(partials, dinv, state, b_conv, W1, b1, W2, b2,
                         W3.reshape(1, -1), b3)
    return act.reshape(-1, ACT_DIM)


# speedup vs baseline: 1.0273x; 1.0273x over previous
import functools

import jax
import jax.numpy as jnp
from jax import lax
from jax.experimental import pallas as pl
from jax.experimental.pallas import tpu as pltpu
from jax.experimental.pallas import tpu_sc as plsc

NC = 2
NS = 16
NW = NC * NS
CH = 128
TRASH = 64
LANES = 16
ACT_DIM = 8


def _leaky_relu(x):
    return jnp.where(x > 0, x, 0.01 * x)


@functools.lru_cache(maxsize=None)
def _make_sc_deg(n, nch):
    mesh = plsc.VectorSubcoreMesh(core_axis_name="c", subcore_axis_name="s",
                                  num_cores=NC, num_subcores=NS)
    zch = n // 5

    @functools.partial(
        pl.kernel,
        out_type=jax.ShapeDtypeStruct((NC * n,), jnp.float32),
        mesh=mesh,
        compiler_params=pltpu.CompilerParams(use_tc_tiling_on_sc=False),
        scratch_types=[
            pltpu.VMEM((nch, CH), jnp.int32),
            pltpu.VMEM((CH,), jnp.float32),
            pltpu.VMEM((zch,), jnp.float32),
            pltpu.VMEM_SHARED((n + TRASH,), jnp.float32),
            pltpu.SemaphoreType.DMA,
        ],
    )
    def deg_kernel(dst_hbm, out_hbm, dst_v, ones_v, zb_v, deg_sh, sem):
        cid = lax.axis_index("c")
        sid = lax.axis_index("s")
        pltpu.sync_copy(dst_hbm.at[cid, sid], dst_v)

        one16 = jnp.full((LANES,), 1.0, jnp.float32)
        zero16 = jnp.zeros((LANES,), jnp.float32)

        def fill_ones(i, c):
            ones_v[pl.ds(pl.multiple_of(i * LANES, LANES), LANES)] = one16
            return c
        lax.fori_loop(0, CH // LANES, fill_ones, 0)

        def fill_zeros(i, c):
            zb_v[pl.ds(pl.multiple_of(i * LANES, LANES), LANES)] = zero16
            return c
        lax.fori_loop(0, zch // LANES, fill_zeros, 0)

        @pl.when(sid < 5)
        def _():
            pltpu.sync_copy(zb_v, deg_sh.at[pl.ds(sid * zch, zch)])
        plsc.subcore_barrier()

        def body(j, c):
            pltpu.async_copy(ones_v, deg_sh.at[dst_v.at[j]], sem, add=True)
            return c
        lax.fori_loop(0, nch, body, 0)

        def drain(j, c):
            pltpu.make_async_copy(ones_v, deg_sh.at[dst_v.at[j]], sem).wait()
            return c
        lax.fori_loop(0, nch, drain, 0)
        plsc.subcore_barrier()

        @pl.when(sid < 5)
        def _():
            pltpu.sync_copy(deg_sh.at[pl.ds(sid * zch, zch)], zb_v)
            pltpu.sync_copy(zb_v, out_hbm.at[pl.ds(cid * n + sid * zch, zch)])

    return deg_kernel


@functools.lru_cache(maxsize=None)
def _make_sc_msg(n, d, nch):
    hd = d // 2
    mesh = plsc.VectorSubcoreMesh(core_axis_name="c", subcore_axis_name="s",
                                  num_cores=NC, num_subcores=NS)
    io_rows = n // 10
    zrows = io_rows // 5
    NB = 4
    assert nch >= NB

    @functools.partial(
        pl.kernel,
        out_type=jax.ShapeDtypeStruct((NC, n, hd), jnp.float32),
        mesh=mesh,
        compiler_params=pltpu.CompilerParams(use_tc_tiling_on_sc=False),
        scratch_types=[
            pltpu.VMEM((nch, CH), jnp.int32),
            pltpu.VMEM((nch, CH), jnp.int32),
            pltpu.VMEM((4, CH, hd), jnp.float32),
            pltpu.VMEM((zrows, hd), jnp.float32),
            pltpu.VMEM_SHARED((n + TRASH, hd), jnp.float32),
            pltpu.SemaphoreType.DMA,
            pltpu.SemaphoreType.DMA,
            pltpu.SemaphoreType.DMA,
            pltpu.SemaphoreType.DMA,
            pltpu.SemaphoreType.DMA,
            pltpu.SemaphoreType.DMA,
            pltpu.SemaphoreType.DMA,
            pltpu.SemaphoreType.DMA,
        ],
    )
    def msg_kernel(y2_hbm, src_hbm, dst_hbm, out_hbm,
                   src_v, dst_v, rows_v, zb_v, acc_sh,
                   sg0, sg1, sg2, sg3, ss0, ss1, ss2, ss3):
        cid = lax.axis_index("c")
        sid = lax.axis_index("s")
        pltpu.sync_copy(src_hbm.at[sid], src_v)
        pltpu.sync_copy(dst_hbm.at[sid], dst_v)

        zero16 = jnp.zeros((LANES,), jnp.float32)
        dl = hd // LANES

        def zrow(i, c):
            r = i // dl
            col = i % dl
            zb_v[r, pl.ds(pl.multiple_of(col * LANES, LANES), LANES)] = zero16
            return c
        lax.fori_loop(0, zrows * dl, zrow, 0)

        @pl.when(sid < 10)
        def _():
            def zcopy(k, c):
                pltpu.sync_copy(zb_v,
                                acc_sh.at[pl.ds(sid * io_rows + k * zrows,
                                                zrows)])
                return c
            lax.fori_loop(0, io_rows // zrows, zcopy, 0)
        plsc.subcore_barrier()

        sems_g = (sg0, sg1, sg2, sg3)
        sems_s = (ss0, ss1, ss2, ss3)

        def g_start(j, b):
            pltpu.async_copy(y2_hbm.at[cid].at[src_v.at[j]], rows_v.at[b],
                             sems_g[b])

        def g_wait(j, b):
            pltpu.make_async_copy(y2_hbm.at[cid].at[src_v.at[j]],
                                  rows_v.at[b], sems_g[b]).wait()

        def s_start(j, b):
            pltpu.async_copy(rows_v.at[b], acc_sh.at[dst_v.at[j]], sems_s[b],
                             add=True)

        def s_wait(j, b):
            pltpu.make_async_copy(rows_v.at[b], acc_sh.at[dst_v.at[j]],
                                  sems_s[b]).wait()

        def visit(j, b):
            g_wait(j, b)
            s_start(j, b)
            jm = j - 2

            @pl.when(jm >= 0)
            def _():
                s_wait(jm, (b + 2) % NB)
            nxt = j + 2

            @pl.when(nxt < nch)
            def _():
                g_start(nxt, (b + 2) % NB)

        g_start(0, 0)
        g_start(1, 1)
        full = nch // NB

        def body(i, c):
            for b in range(NB):
                visit(i * NB + b, b)
            return c
        lax.fori_loop(0, full, body, 0)
        for t in range(full * NB, nch):
            visit(t, t % NB)
        s_wait(nch - 2, (nch - 2) % NB)
        s_wait(nch - 1, (nch - 1) % NB)
        plsc.subcore_barrier()

        @pl.when(sid < 10)
        def _():
            def wcopy(k, c):
                off = sid * io_rows + k * zrows
                pltpu.sync_copy(acc_sh.at[pl.ds(off, zrows)], zb_v)
                pltpu.sync_copy(zb_v, out_hbm.at[cid, pl.ds(off, zrows)])
                return c
            lax.fori_loop(0, io_rows // zrows, wcopy, 0)

    return msg_kernel


def _tc_prep_call(cnt, state, w_conv):
    n, d = state.shape
    hd = d // 2

    def prep(cnt_ref, x_ref, w_ref, y2_ref, dinv_ref):
        deg = cnt_ref[0] + cnt_ref[1] + 1.0
        dinv = lax.rsqrt(deg)
        xw = jnp.dot(x_ref[...], w_ref[...], preferred_element_type=jnp.float32)
        y = xw * dinv[:, None]
        y2_ref[0] = y[:, :hd]
        y2_ref[1] = y[:, hd:]
        dinv_ref[...] = dinv

    return pl.pallas_call(
        prep,
        out_shape=[jax.ShapeDtypeStruct((NC, n, hd), jnp.float32),
                   jax.ShapeDtypeStruct((n,), jnp.float32)],
    )(cnt, state, w_conv)


def _tc_final_call(partials, dinv, state, b_conv, w1, b1, w2, b2, w3r, b3):
    n, d = state.shape

    def fin(p_ref, dinv_ref, x_ref, bc_ref, w1_ref, b1_ref, w2_ref,
            b2_ref, w3_ref, b3_ref, out_ref):
        acc = jnp.concatenate([p_ref[0], p_ref[1]], axis=-1)
        conv = acc * dinv_ref[...][:, None] + bc_ref[...][None, :]
        x = jnp.maximum(conv, 0.0) + x_ref[...]
        h1 = _leaky_relu(jnp.dot(x, w1_ref[...],
                                 preferred_element_type=jnp.float32)
                         + b1_ref[...][None, :])
        h2 = _leaky_relu(jnp.dot(h1, w2_ref[...],
                                 preferred_element_type=jnp.float32)
                         + b2_ref[...][None, :])
        logit = jnp.sum(h2 * w3_ref[...], axis=1) + b3_ref[0]
        conc = jnp.log(1.0 + jnp.exp(-jnp.abs(logit))) + jnp.maximum(logit, 0.0)
        out_ref[...] = conc / (jnp.sum(conc) + 1e-20)

    return pl.pallas_call(
        fin,
        out_shape=jax.ShapeDtypeStruct((n,), jnp.float32),
    )(partials, dinv, state, b_conv, w1, b1, w2, b2, w3r, b3)


def kernel(state, edge_index, W_conv, b_conv, W1, b1, W2, b2, W3, b3,
           deterministic):
    n, d = state.shape
    e = edge_index.shape[1]

    def pad_edges(arr, parts):
        srcp, dstp = arr
        per = srcp.shape[0] // parts
        nchunks = -(-per // CH)
        pad = nchunks * CH - per
        srcp = srcp.reshape(parts, per)
        dstp = dstp.reshape(parts, per)
        if pad:
            k = jnp.arange(pad, dtype=jnp.int32)
            ps = jnp.broadcast_to((k * 131) % n, (parts, pad))
            pd = jnp.broadcast_to(n + (k % TRASH), (parts, pad))
            srcp = jnp.concatenate([srcp, ps], axis=1)
            dstp = jnp.concatenate([dstp, pd], axis=1)
        return (srcp.reshape(parts, nchunks, CH),
                dstp.reshape(parts, nchunks, CH), nchunks)

    _, dstd, nch = pad_edges((edge_index[0], edge_index[1]), NW)
    loop = jnp.arange(n, dtype=edge_index.dtype)
    srcl = jnp.concatenate([edge_index[0], loop])
    dstl = jnp.concatenate([edge_index[1], loop])
    src2, dst2, nch2 = pad_edges((srcl, dstl), NS)
    dstd = dstd.reshape(NC, NS, nch, CH)

    cnt = _make_sc_deg(n, nch)(dstd).reshape(NC, n)
    y2, dinv = _tc_prep_call(cnt, state, W_conv)
    partials = _make_sc_msg(n, d, nch2)(y2, src2, dst2)
    act = _tc_final_call(partials, dinv, state, b_conv, W1, b1, W2, b2,
                         W3.reshape(1, -1), b3)
    return act.reshape(-1, ACT_DIM)


# speedup vs baseline: 1.0528x; 1.0249x over previous
import functools

import jax
import jax.numpy as jnp
from jax import lax
from jax.experimental import pallas as pl
from jax.experimental.pallas import tpu as pltpu
from jax.experimental.pallas import tpu_sc as plsc

NC = 2
NS = 16
NW = NC * NS
CH = 128
TRASH = 64
LANES = 16
ACT_DIM = 8


def _leaky_relu(x):
    return jnp.where(x > 0, x, 0.01 * x)


@functools.lru_cache(maxsize=None)
def _make_sc_deg(n, nch):
    mesh = plsc.VectorSubcoreMesh(core_axis_name="c", subcore_axis_name="s",
                                  num_cores=NC, num_subcores=NS)
    zch = n // 5

    @functools.partial(
        pl.kernel,
        out_type=jax.ShapeDtypeStruct((NC * n,), jnp.float32),
        mesh=mesh,
        compiler_params=pltpu.CompilerParams(use_tc_tiling_on_sc=False),
        scratch_types=[
            pltpu.VMEM((nch, CH), jnp.int32),
            pltpu.VMEM((CH,), jnp.float32),
            pltpu.VMEM((zch,), jnp.float32),
            pltpu.VMEM_SHARED((n + TRASH,), jnp.float32),
            pltpu.SemaphoreType.DMA,
        ],
    )
    def deg_kernel(dst_hbm, out_hbm, dst_v, ones_v, zb_v, deg_sh, sem):
        cid = lax.axis_index("c")
        sid = lax.axis_index("s")
        pltpu.sync_copy(dst_hbm.at[cid, sid], dst_v)

        one16 = jnp.full((LANES,), 1.0, jnp.float32)
        zero16 = jnp.zeros((LANES,), jnp.float32)

        def fill_ones(i, c):
            ones_v[pl.ds(pl.multiple_of(i * LANES, LANES), LANES)] = one16
            return c
        lax.fori_loop(0, CH // LANES, fill_ones, 0)

        def fill_zeros(i, c):
            zb_v[pl.ds(pl.multiple_of(i * LANES, LANES), LANES)] = zero16
            return c
        lax.fori_loop(0, zch // LANES, fill_zeros, 0)

        @pl.when(sid < 5)
        def _():
            pltpu.sync_copy(zb_v, deg_sh.at[pl.ds(sid * zch, zch)])
        plsc.subcore_barrier()

        def body(j, c):
            pltpu.async_copy(ones_v, deg_sh.at[dst_v.at[j]], sem, add=True)
            return c
        lax.fori_loop(0, nch, body, 0)

        def drain(j, c):
            pltpu.make_async_copy(ones_v, deg_sh.at[dst_v.at[j]], sem).wait()
            return c
        lax.fori_loop(0, nch, drain, 0)
        plsc.subcore_barrier()

        @pl.when(sid < 5)
        def _():
            pltpu.sync_copy(deg_sh.at[pl.ds(sid * zch, zch)], zb_v)
            pltpu.sync_copy(zb_v, out_hbm.at[pl.ds(cid * n + sid * zch, zch)])

    return deg_kernel


@functools.lru_cache(maxsize=None)
def _make_sc_msg(n, d, nch):
    hd = d // 2
    mesh = plsc.VectorSubcoreMesh(core_axis_name="c", subcore_axis_name="s",
                                  num_cores=NC, num_subcores=NS)
    io_rows = n // 10
    zrows = io_rows // 5
    NB = 4
    assert nch >= NB

    @functools.partial(
        pl.kernel,
        out_type=jax.ShapeDtypeStruct((NC, n, hd), jnp.float32),
        mesh=mesh,
        compiler_params=pltpu.CompilerParams(use_tc_tiling_on_sc=False),
        scratch_types=[
            pltpu.VMEM((nch, CH), jnp.int32),
            pltpu.VMEM((nch, CH), jnp.int32),
            pltpu.VMEM((4, CH, hd), jnp.float32),
            pltpu.VMEM((zrows, hd), jnp.float32),
            pltpu.VMEM_SHARED((n + TRASH, hd), jnp.float32),
            pltpu.SemaphoreType.DMA,
            pltpu.SemaphoreType.DMA,
            pltpu.SemaphoreType.DMA,
            pltpu.SemaphoreType.DMA,
            pltpu.SemaphoreType.DMA,
            pltpu.SemaphoreType.DMA,
            pltpu.SemaphoreType.DMA,
            pltpu.SemaphoreType.DMA,
        ],
    )
    def msg_kernel(y2_hbm, src_hbm, dst_hbm, out_hbm,
                   src_v, dst_v, rows_v, zb_v, acc_sh,
                   sg0, sg1, sg2, sg3, ss0, ss1, ss2, ss3):
        cid = lax.axis_index("c")
        sid = lax.axis_index("s")
        pltpu.async_copy(src_hbm.at[sid], src_v, sg0)
        pltpu.async_copy(dst_hbm.at[sid], dst_v, sg1)

        zero16 = jnp.zeros((LANES,), jnp.float32)
        dl = hd // LANES

        def zrow(i, c):
            r = i // dl
            col = i % dl
            zb_v[r, pl.ds(pl.multiple_of(col * LANES, LANES), LANES)] = zero16
            return c
        lax.fori_loop(0, zrows * dl, zrow, 0)

        @pl.when(sid < 10)
        def _():
            def zcopy(k, c):
                pltpu.sync_copy(zb_v,
                                acc_sh.at[pl.ds(sid * io_rows + k * zrows,
                                                zrows)])
                return c
            lax.fori_loop(0, io_rows // zrows, zcopy, 0)
        pltpu.make_async_copy(src_hbm.at[sid], src_v, sg0).wait()
        pltpu.make_async_copy(dst_hbm.at[sid], dst_v, sg1).wait()
        plsc.subcore_barrier()

        sems_g = (sg0, sg1, sg2, sg3)
        sems_s = (ss0, ss1, ss2, ss3)

        def g_start(j, b):
            pltpu.async_copy(y2_hbm.at[cid].at[src_v.at[j]], rows_v.at[b],
                             sems_g[b])

        def g_wait(j, b):
            pltpu.make_async_copy(y2_hbm.at[cid].at[src_v.at[j]],
                                  rows_v.at[b], sems_g[b]).wait()

        def s_start(j, b):
            pltpu.async_copy(rows_v.at[b], acc_sh.at[dst_v.at[j]], sems_s[b],
                             add=True)

        def s_wait(j, b):
            pltpu.make_async_copy(rows_v.at[b], acc_sh.at[dst_v.at[j]],
                                  sems_s[b]).wait()

        def visit(j, b):
            g_wait(j, b)
            s_start(j, b)
            jm = j - 2

            @pl.when(jm >= 0)
            def _():
                s_wait(jm, (b + 2) % NB)
            nxt = j + 2

            @pl.when(nxt < nch)
            def _():
                g_start(nxt, (b + 2) % NB)

        g_start(0, 0)
        g_start(1, 1)
        full = nch // NB

        def body(i, c):
            for b in range(NB):
                visit(i * NB + b, b)
            return c
        lax.fori_loop(0, full, body, 0)
        for t in range(full * NB, nch):
            visit(t, t % NB)
        s_wait(nch - 2, (nch - 2) % NB)
        s_wait(nch - 1, (nch - 1) % NB)
        plsc.subcore_barrier()

        @pl.when(sid < 10)
        def _():
            wrows = CH - 3
            nw = io_rows // wrows
            bufs = (rows_v.at[0, pl.ds(0, wrows)], rows_v.at[1, pl.ds(0, wrows)])
            for k in range(nw):
                off = sid * io_rows + k * wrows
                b = bufs[k % 2]
                if k >= 2:
                    offp = sid * io_rows + (k - 2) * wrows
                    pltpu.make_async_copy(
                        b, out_hbm.at[cid, pl.ds(offp, wrows)], ss0).wait()
                pltpu.sync_copy(acc_sh.at[pl.ds(off, wrows)], b)
                pltpu.async_copy(b, out_hbm.at[cid, pl.ds(off, wrows)], ss0)
            for k in range(max(nw - 2, 0), nw):
                off = sid * io_rows + k * wrows
                pltpu.make_async_copy(
                    bufs[k % 2], out_hbm.at[cid, pl.ds(off, wrows)], ss0).wait()

    return msg_kernel


def _tc_prep_call(cnt, state, w_conv):
    n, d = state.shape
    hd = d // 2

    def prep(cnt_ref, x_ref, w_ref, y2_ref, dinv_ref):
        deg = cnt_ref[0] + cnt_ref[1] + 1.0
        dinv = lax.rsqrt(deg)
        xw = jnp.dot(x_ref[...], w_ref[...], preferred_element_type=jnp.float32)
        y = xw * dinv[:, None]
        y2_ref[0] = y[:, :hd]
        y2_ref[1] = y[:, hd:]
        dinv_ref[...] = dinv

    return pl.pallas_call(
        prep,
        out_shape=[jax.ShapeDtypeStruct((NC, n, hd), jnp.float32),
                   jax.ShapeDtypeStruct((n,), jnp.float32)],
    )(cnt, state, w_conv)


def _tc_final_call(partials, dinv, state, b_conv, w1, b1, w2, b2, w3r, b3):
    n, d = state.shape

    def fin(p_ref, dinv_ref, x_ref, bc_ref, w1_ref, b1_ref, w2_ref,
            b2_ref, w3_ref, b3_ref, out_ref):
        acc = jnp.concatenate([p_ref[0], p_ref[1]], axis=-1)
        conv = acc * dinv_ref[...][:, None] + bc_ref[...][None, :]
        x = jnp.maximum(conv, 0.0) + x_ref[...]
        h1 = _leaky_relu(jnp.dot(x, w1_ref[...],
                                 preferred_element_type=jnp.float32)
                         + b1_ref[...][None, :])
        h2 = _leaky_relu(jnp.dot(h1, w2_ref[...],
                                 preferred_element_type=jnp.float32)
                         + b2_ref[...][None, :])
        logit = jnp.sum(h2 * w3_ref[...], axis=1) + b3_ref[0]
        conc = jnp.log(1.0 + jnp.exp(-jnp.abs(logit))) + jnp.maximum(logit, 0.0)
        out_ref[...] = conc / (jnp.sum(conc) + 1e-20)

    return pl.pallas_call(
        fin,
        out_shape=jax.ShapeDtypeStruct((n,), jnp.float32),
    )(partials, dinv, state, b_conv, w1, b1, w2, b2, w3r, b3)


def kernel(state, edge_index, W_conv, b_conv, W1, b1, W2, b2, W3, b3,
           deterministic):
    n, d = state.shape
    e = edge_index.shape[1]

    def pad_edges(arr, parts):
        srcp, dstp = arr
        per = srcp.shape[0] // parts
        nchunks = -(-per // CH)
        pad = nchunks * CH - per
        srcp = srcp.reshape(parts, per)
        dstp = dstp.reshape(parts, per)
        if pad:
            k = jnp.arange(pad, dtype=jnp.int32)
            ps = jnp.broadcast_to((k * 131) % n, (parts, pad))
            pd = jnp.broadcast_to(n + (k % TRASH), (parts, pad))
            srcp = jnp.concatenate([srcp, ps], axis=1)
            dstp = jnp.concatenate([dstp, pd], axis=1)
        return (srcp.reshape(parts, nchunks, CH),
                dstp.reshape(parts, nchunks, CH), nchunks)

    _, dstd, nch = pad_edges((edge_index[0], edge_index[1]), NW)
    loop = jnp.arange(n, dtype=edge_index.dtype)
    srcl = jnp.concatenate([edge_index[0], loop])
    dstl = jnp.concatenate([edge_index[1], loop])
    src2, dst2, nch2 = pad_edges((srcl, dstl), NS)
    dstd = dstd.reshape(NC, NS, nch, CH)

    cnt = _make_sc_deg(n, nch)(dstd).reshape(NC, n)
    y2, dinv = _tc_prep_call(cnt, state, W_conv)
    partials = _make_sc_msg(n, d, nch2)(y2, src2, dst2)
    act = _tc_final_call(partials, dinv, state, b_conv, W1, b1, W2, b2,
                         W3.reshape(1, -1), b3)
    return act.reshape(-1, ACT_DIM)
